# speedup vs baseline: 6.6527x; 6.6527x over previous
import functools

import jax
import jax.numpy as jnp
from jax import lax
from jax.experimental import pallas as pl
from jax.experimental.pallas import tpu as pltpu
from jax.experimental.pallas import tpu_sc as plsc

VOCAB = 1000
FEAT = 128
LPG = 128
K = 2
CHUNK = K * LPG


@functools.cache
def _build(B):
    info = plsc.get_sparse_core_info()
    NW = info.num_cores * info.num_subcores
    per_w = B // NW
    n_chunks = per_w // CHUNK
    assert per_w % CHUNK == 0
    rows_per_w = per_w // LPG

    mesh = plsc.VectorSubcoreMesh(core_axis_name="c", subcore_axis_name="s")

    @functools.partial(
        pl.kernel,
        out_type=jax.ShapeDtypeStruct((B, FEAT), jnp.float32),
        mesh=mesh,
        scratch_types=[
            pltpu.VMEM((K, LPG), jnp.int32),
            pltpu.VMEM((CHUNK, FEAT), jnp.float32),
            pltpu.SemaphoreType.DMA,
            pltpu.SemaphoreType.DMA,
        ],
    )
    def k(idx_hbm, table_hbm, out_hbm, idx_v, rows_v, sem_g, sem_o):
        wid = lax.axis_index("s") * info.num_cores + lax.axis_index("c")
        row0 = wid * rows_per_w
        out0 = wid * per_w

        def body(g, _):
            pltpu.sync_copy(idx_hbm.at[pl.ds(row0 + g * K, K)], idx_v)
            copies = [
                pltpu.async_copy(
                    table_hbm.at[idx_v.at[j]],
                    rows_v.at[pl.ds(j * LPG, LPG)],
                    sem_g,
                )
                for j in range(K)
            ]
            for c in copies:
                c.wait()
            pltpu.async_copy(
                rows_v, out_hbm.at[pl.ds(out0 + g * CHUNK, CHUNK)], sem_o
            ).wait()
            return ()

        lax.fori_loop(0, n_chunks, body, (), unroll=False)

    return k


def kernel(word_indexs, embedding_weight):
    B = word_indexs.shape[0] * word_indexs.shape[1]
    idx2d = word_indexs.reshape(B // LPG, LPG).astype(jnp.int32)
    out = _build(B)(idx2d, embedding_weight)
    return out.reshape(word_indexs.shape[0], word_indexs.shape[1], FEAT)


# speedup vs baseline: 6.9526x; 1.0451x over previous
import functools

import jax
import jax.numpy as jnp
from jax import lax
from jax.experimental import pallas as pl
from jax.experimental.pallas import tpu as pltpu
from jax.experimental.pallas import tpu_sc as plsc

VOCAB = 1000
FEAT = 128
LPG = 128
K = 2
CHUNK = K * LPG
NBUF = 2


@functools.cache
def _build(B):
    info = plsc.get_sparse_core_info()
    NW = info.num_cores * info.num_subcores
    per_w = B // NW
    n_chunks = per_w // CHUNK
    assert per_w % CHUNK == 0 and n_chunks % NBUF == 0 and n_chunks >= 2 * NBUF
    rows_per_w = per_w // LPG

    mesh = plsc.VectorSubcoreMesh(core_axis_name="c", subcore_axis_name="s")

    @functools.partial(
        pl.kernel,
        out_type=jax.ShapeDtypeStruct((B, FEAT), jnp.float32),
        mesh=mesh,
        scratch_types=[
            pltpu.VMEM((NBUF, K, LPG), jnp.int32),
            pltpu.VMEM((NBUF, CHUNK, FEAT), jnp.float32),
            pltpu.SemaphoreType.DMA,
            pltpu.SemaphoreType.DMA,
            pltpu.SemaphoreType.DMA,
            pltpu.SemaphoreType.DMA,
            pltpu.SemaphoreType.DMA,
        ],
    )
    def k(idx_hbm, table_hbm, out_hbm, idx_v, rows_v,
          sem_i0, sem_i1, sem_g, sem_o0, sem_o1):
        sem_i = (sem_i0, sem_i1)
        sem_o = (sem_o0, sem_o1)
        wid = lax.axis_index("s") * info.num_cores + lax.axis_index("c")
        row0 = wid * rows_per_w
        out0 = wid * per_w

        def idx_copy(g, b):
            return pltpu.make_async_copy(
                idx_hbm.at[pl.ds(row0 + g * K, K)], idx_v.at[b], sem_i[b]
            )

        def out_copy(g, b):
            return pltpu.make_async_copy(
                rows_v.at[b], out_hbm.at[pl.ds(out0 + g * CHUNK, CHUNK)],
                sem_o[b],
            )

        for b in range(NBUF):
            idx_copy(b, b).start()

        @pl.loop(0, n_chunks, step=NBUF)
        def _(g0):
            for b in range(NBUF):
                g = g0 + b

                @pl.when(g >= NBUF)
                def _():
                    out_copy(g - NBUF, b).wait()

                idx_copy(g, b).wait()
                gathers = [
                    pltpu.async_copy(
                        table_hbm.at[idx_v.at[b, j]],
                        rows_v.at[b, pl.ds(j * LPG, LPG)],
                        sem_g,
                    )
                    for j in range(K)
                ]
                for c in gathers:
                    c.wait()
                out_copy(g, b).start()

                @pl.when(g + NBUF < n_chunks)
                def _():
                    idx_copy(g + NBUF, b).start()

        for b in range(NBUF):
            out_copy(n_chunks - NBUF + b, b).wait()

    return k


def kernel(word_indexs, embedding_weight):
    B = word_indexs.shape[0] * word_indexs.shape[1]
    idx2d = word_indexs.reshape(B // LPG, LPG).astype(jnp.int32)
    out = _build(B)(idx2d, embedding_weight)
    return out.reshape(word_indexs.shape[0], word_indexs.shape[1], FEAT)


# speedup vs baseline: 18.9633x; 2.7275x over previous
import functools

import jax
import jax.numpy as jnp
from jax import lax
from jax.experimental import pallas as pl
from jax.experimental.pallas import tpu as pltpu
from jax.experimental.pallas import tpu_sc as plsc

VOCAB = 1000
VPAD = 1024
FEAT = 128
LPG = 128
K = 2
CHUNK = K * LPG
NBUF = 2


@functools.cache
def _build(B):
    info = plsc.get_sparse_core_info()
    NW = info.num_cores * info.num_subcores
    per_w = B // NW
    n_chunks = per_w // CHUNK
    assert per_w % CHUNK == 0 and n_chunks % NBUF == 0 and n_chunks >= 2 * NBUF
    rows_per_w = per_w // LPG

    mesh = plsc.VectorSubcoreMesh(core_axis_name="c", subcore_axis_name="s")

    @functools.partial(
        pl.kernel,
        out_type=jax.ShapeDtypeStruct((B, FEAT), jnp.float32),
        mesh=mesh,
        scratch_types=[
            pltpu.VMEM((NBUF, K, LPG), jnp.int32),
            pltpu.VMEM((NBUF, CHUNK, FEAT), jnp.float32),
            pltpu.VMEM_SHARED((VPAD, FEAT), jnp.float32),
            pltpu.SemaphoreType.DMA,
            pltpu.SemaphoreType.DMA,
            pltpu.SemaphoreType.DMA,
            pltpu.SemaphoreType.DMA,
            pltpu.SemaphoreType.DMA,
        ],
    )
    def k(idx_hbm, table_hbm, out_hbm, idx_v, rows_v, table_sp,
          sem_i0, sem_i1, sem_g, sem_o0, sem_o1):
        sem_i = (sem_i0, sem_i1)
        sem_o = (sem_o0, sem_o1)
        wid = lax.axis_index("s") * info.num_cores + lax.axis_index("c")
        row0 = wid * rows_per_w
        out0 = wid * per_w

        def idx_copy(g, b):
            return pltpu.make_async_copy(
                idx_hbm.at[pl.ds(row0 + g * K, K)], idx_v.at[b], sem_i[b]
            )

        def out_copy(g, b):
            return pltpu.make_async_copy(
                rows_v.at[b], out_hbm.at[pl.ds(out0 + g * CHUNK, CHUNK)],
                sem_o[b],
            )

        sub = lax.axis_index("s")
        tslice = pl.ds(sub * (VPAD // 16), VPAD // 16)
        pltpu.sync_copy(table_hbm.at[tslice], rows_v.at[0, pl.ds(0, VPAD // 16)])
        pltpu.sync_copy(rows_v.at[0, pl.ds(0, VPAD // 16)], table_sp.at[tslice])
        plsc.subcore_barrier()

        for b in range(NBUF):
            idx_copy(b, b).start()

        @pl.loop(0, n_chunks, step=NBUF)
        def _(g0):
            for b in range(NBUF):
                g = g0 + b

                @pl.when(g >= NBUF)
                def _():
                    out_copy(g - NBUF, b).wait()

                idx_copy(g, b).wait()
                gathers = [
                    pltpu.async_copy(
                        table_sp.at[idx_v.at[b, j]],
                        rows_v.at[b, pl.ds(j * LPG, LPG)],
                        sem_g,
                    )
                    for j in range(K)
                ]
                for c in gathers:
                    c.wait()
                out_copy(g, b).start()

                @pl.when(g + NBUF < n_chunks)
                def _():
                    idx_copy(g + NBUF, b).start()

        for b in range(NBUF):
            out_copy(n_chunks - NBUF + b, b).wait()

    return k


def kernel(word_indexs, embedding_weight):
    B = word_indexs.shape[0] * word_indexs.shape[1]
    idx2d = word_indexs.reshape(B // LPG, LPG).astype(jnp.int32)
    tpad = jnp.pad(embedding_weight, ((0, VPAD - VOCAB), (0, 0)))
    out = _build(B)(idx2d, tpad)
    return out.reshape(word_indexs.shape[0], word_indexs.shape[1], FEAT)
